# Initial kernel scaffold; baseline (speedup 1.0000x reference)
import jax
import jax.numpy as jnp
from jax.experimental import pallas as pl


def kernel(x, edge_index, edge_attr, Wn0, We0, Wfn0, bfn0, Wfe0, bfe0, Wn1, We1, Wfn1, bfn1, Wfe1, bfe1):
    raise NotImplementedError("write your pallas kernel here")


# speedup vs baseline: 1.1621x; 1.1621x over previous
import jax
import jax.numpy as jnp
from jax.experimental import pallas as pl
from jax.experimental.pallas import tpu as pltpu

_C = 1.0 / (1.0 + 1e-5) ** 0.5


def _h0_body(x_ref, wn0_ref, h0_ref):
    h0_ref[...] = jax.nn.relu(x_ref[...] @ wn0_ref[...])


def _edge_body(ea_ref, s_ref, we0_ref, wfa_ref, wfb_ref, bfe0_ref, we1_ref,
               tw_ref):
    t = jax.nn.relu(ea_ref[...] @ we0_ref[...])
    v = jax.nn.relu(s_ref[...] @ wfa_ref[...] + t @ wfb_ref[...]
                    + bfe0_ref[...])
    w = jax.nn.relu(v @ we1_ref[...])
    tw_ref[...] = jnp.concatenate([t, w], axis=1)


def _node_body(aggr_ref, h0_ref, wfn0a_ref, wfn0b_ref, bfn0_ref, wn1_ref,
               wfn1a_ref, wfn1b_ref, bfn1_ref, out_ref):
    aggr = aggr_ref[...]
    node1 = jax.nn.relu(aggr[:, :64] @ wfn0a_ref[...]
                        + h0_ref[...] @ wfn0b_ref[...] + bfn0_ref[...]) * _C
    h1 = jax.nn.relu(node1 @ wn1_ref[...])
    out_ref[...] = jax.nn.relu(aggr[:, 64:] @ wfn1a_ref[...]
                               + h1 @ wfn1b_ref[...] + bfn1_ref[...]) * _C


def _full(shape):
    return pl.BlockSpec(shape, lambda i: (0,) * len(shape))


def kernel(x, edge_index, edge_attr, Wn0, We0, Wfn0, bfn0, Wfe0, bfe0,
           Wn1, We1, Wfn1, bfn1, Wfe1, bfe1):
    N, DN = x.shape
    E, DE = edge_attr.shape
    H = Wfn0.shape[0]
    Hh = H // 2
    src = edge_index[0]
    dst = edge_index[1]

    bn = 2000 if N % 2000 == 0 else N
    be = 4000 if E % 4000 == 0 else E

    h0 = pl.pallas_call(
        _h0_body,
        grid=(N // bn,),
        in_specs=[pl.BlockSpec((bn, DN), lambda i: (i, 0)), _full((DN, Hh))],
        out_specs=pl.BlockSpec((bn, Hh), lambda i: (i, 0)),
        out_shape=jax.ShapeDtypeStruct((N, Hh), jnp.float32),
    )(x, Wn0)

    s = h0[src] + h0[dst]

    tw = pl.pallas_call(
        _edge_body,
        grid=(E // be,),
        in_specs=[
            pl.BlockSpec((be, DE), lambda i: (i, 0)),
            pl.BlockSpec((be, Hh), lambda i: (i, 0)),
            _full((DE, Hh)),
            _full((Hh, H)),
            _full((Hh, H)),
            _full((1, H)),
            _full((H, Hh)),
        ],
        out_specs=pl.BlockSpec((be, H), lambda i: (i, 0)),
        out_shape=jax.ShapeDtypeStruct((E, H), jnp.float32),
    )(edge_attr, s, We0, Wfe0[:Hh], Wfe0[Hh:], bfe0[None], We1)

    aggr = jax.ops.segment_sum(tw, dst, num_segments=N)

    out = pl.pallas_call(
        _node_body,
        grid=(N // bn,),
        in_specs=[
            pl.BlockSpec((bn, H), lambda i: (i, 0)),
            pl.BlockSpec((bn, Hh), lambda i: (i, 0)),
            _full((Hh, H)),
            _full((Hh, H)),
            _full((1, H)),
            _full((H, Hh)),
            _full((Hh, H)),
            _full((Hh, H)),
            _full((1, H)),
        ],
        out_specs=pl.BlockSpec((bn, H), lambda i: (i, 0)),
        out_shape=jax.ShapeDtypeStruct((N, H), jnp.float32),
    )(aggr, h0, Wfn0[:Hh], Wfn0[Hh:], bfn0[None], Wn1,
      Wfn1[:Hh], Wfn1[Hh:], bfn1[None])

    return out


# speedup vs baseline: 3.6451x; 3.1366x over previous
import functools

import jax
import jax.numpy as jnp
from jax import lax
from jax.experimental import pallas as pl
from jax.experimental.pallas import tpu as pltpu
from jax.experimental.pallas import tpu_sc as plsc

_C = 1.0 / (1.0 + 1e-5) ** 0.5

_NC = 2
_NS = 16
_NW = _NC * _NS


def _h0_body(x_ref, wn0_ref, h0_ref):
    h0_ref[...] = jax.nn.relu(x_ref[...] @ wn0_ref[...])


def _edge_body(ea_ref, sa_ref, sb_ref, we0_ref, wfa_ref, wfb_ref, bfe0_ref,
               we1_ref, tw_ref):
    t = jax.nn.relu(ea_ref[...] @ we0_ref[...])
    m = sa_ref[...] + sb_ref[...]
    v = jax.nn.relu(m @ wfa_ref[...] + t @ wfb_ref[...]
                    + bfe0_ref[...])
    w = jax.nn.relu(v @ we1_ref[...])
    tw_ref[...] = jnp.concatenate([t, w], axis=1)


def _node_body(ap_ref, h0_ref, wfn0a_ref, wfn0b_ref, bfn0_ref, wn1_ref,
               wfn1a_ref, wfn1b_ref, bfn1_ref, out_ref):
    aggr = ap_ref[0] + ap_ref[1]
    node1 = jax.nn.relu(aggr[:, :64] @ wfn0a_ref[...]
                        + h0_ref[...] @ wfn0b_ref[...] + bfn0_ref[...]) * _C
    h1 = jax.nn.relu(node1 @ wn1_ref[...])
    out_ref[...] = jax.nn.relu(aggr[:, 64:] @ wfn1a_ref[...]
                               + h1 @ wfn1b_ref[...] + bfn1_ref[...]) * _C


def _full(shape):
    return pl.BlockSpec(shape, lambda i: (0,) * len(shape))


def _make_sc_gather(N, E, Hh):
    B = 400
    per_w = E // _NW
    n_chunks = per_w // B
    assert per_w % B == 0 and E % _NW == 0

    mesh = plsc.VectorSubcoreMesh(core_axis_name="c", subcore_axis_name="s",
                                  num_cores=_NC, num_subcores=_NS)

    @functools.partial(
        pl.kernel,
        out_type=[jax.ShapeDtypeStruct((E, Hh), jnp.float32),
                  jax.ShapeDtypeStruct((E, Hh), jnp.float32)],
        mesh=mesh,
        compiler_params=pltpu.CompilerParams(use_tc_tiling_on_sc=False),
        scratch_types=[
            pltpu.VMEM((B,), jnp.int32),
            pltpu.VMEM((B,), jnp.int32),
            pltpu.VMEM((B, Hh), jnp.float32),
            pltpu.VMEM((B, Hh), jnp.float32),
            pltpu.SemaphoreType.DMA,
            pltpu.SemaphoreType.DMA,
        ],
    )
    def gather(h0_hbm, src_hbm, dst_hbm, sa_hbm, sb_hbm,
               idxa, idxb, rowsa, rowsb, sema, semb):
        w = lax.axis_index("s") * _NC + lax.axis_index("c")
        base = w * per_w

        def body(j, carry):
            off = base + j * B
            pltpu.sync_copy(src_hbm.at[pl.ds(off, B)], idxa)
            pltpu.sync_copy(dst_hbm.at[pl.ds(off, B)], idxb)
            cpa = pltpu.async_copy(h0_hbm.at[idxa], rowsa, sema)
            cpb = pltpu.async_copy(h0_hbm.at[idxb], rowsb, semb)
            cpa.wait()
            cpb.wait()
            pltpu.sync_copy(rowsa, sa_hbm.at[pl.ds(off, B)])
            pltpu.sync_copy(rowsb, sb_hbm.at[pl.ds(off, B)])
            return carry

        lax.fori_loop(0, n_chunks, body, 0)

    return gather


def _make_sc_scatter(N, E, H):
    R = E // 128
    per_w = R // _NW
    rem = R - per_w * _NW
    K = 3
    n_super = per_w // K
    tail = per_w - n_super * K
    rows_per_sub = N // _NS

    mesh = plsc.VectorSubcoreMesh(core_axis_name="c", subcore_axis_name="s",
                                  num_cores=_NC, num_subcores=_NS)

    @functools.partial(
        pl.kernel,
        out_type=jax.ShapeDtypeStruct((_NC, N, H), jnp.float32),
        mesh=mesh,
        compiler_params=pltpu.CompilerParams(use_tc_tiling_on_sc=False),
        scratch_types=[
            pltpu.VMEM((K, 128), jnp.int32),
            pltpu.VMEM((K * 128, H), jnp.float32),
            pltpu.VMEM_SHARED((N, H), jnp.float32),
        ],
    )
    def scatter(tw_hbm, dst2d_hbm, zeros_hbm, out_hbm, idxk, twbuf, acc):
        c = lax.axis_index("c")
        sid = lax.axis_index("s")
        w = sid * _NC + c

        pltpu.sync_copy(zeros_hbm,
                        acc.at[pl.ds(sid * rows_per_sub, rows_per_sub)])
        plsc.subcore_barrier()

        def do_rows(r0, nrows):
            pltpu.sync_copy(dst2d_hbm.at[pl.ds(r0, nrows)],
                            idxk.at[pl.ds(0, nrows)])
            pltpu.sync_copy(tw_hbm.at[pl.ds(r0 * 128, nrows * 128)],
                            twbuf.at[pl.ds(0, nrows * 128)])
            for t in range(nrows):
                pltpu.sync_copy(twbuf.at[pl.ds(t * 128, 128)],
                                acc.at[idxk.at[t]], add=True)

        def body(j, carry):
            do_rows(w * per_w + j * K, K)
            return carry

        lax.fori_loop(0, n_super, body, 0)
        if tail:
            do_rows(w * per_w + n_super * K, tail)

        @pl.when(w < rem)
        def _():
            do_rows(_NW * per_w + w, 1)

        plsc.subcore_barrier()
        pltpu.sync_copy(acc.at[pl.ds(sid * rows_per_sub, rows_per_sub)],
                        out_hbm.at[c, pl.ds(sid * rows_per_sub, rows_per_sub)])

    return scatter


def kernel(x, edge_index, edge_attr, Wn0, We0, Wfn0, bfn0, Wfe0, bfe0,
           Wn1, We1, Wfn1, bfn1, Wfe1, bfe1):
    N, DN = x.shape
    E, DE = edge_attr.shape
    H = Wfn0.shape[0]
    Hh = H // 2
    src = edge_index[0]
    dst = edge_index[1]

    bn = 2000 if N % 2000 == 0 else N
    be = 4000 if E % 4000 == 0 else E

    h0 = pl.pallas_call(
        _h0_body,
        grid=(N // bn,),
        in_specs=[pl.BlockSpec((bn, DN), lambda i: (i, 0)), _full((DN, Hh))],
        out_specs=pl.BlockSpec((bn, Hh), lambda i: (i, 0)),
        out_shape=jax.ShapeDtypeStruct((N, Hh), jnp.float32),
    )(x, Wn0)

    sa, sb = _make_sc_gather(N, E, Hh)(h0, src, dst)

    tw = pl.pallas_call(
        _edge_body,
        grid=(E // be,),
        in_specs=[
            pl.BlockSpec((be, DE), lambda i: (i, 0)),
            pl.BlockSpec((be, Hh), lambda i: (i, 0)),
            pl.BlockSpec((be, Hh), lambda i: (i, 0)),
            _full((DE, Hh)),
            _full((Hh, H)),
            _full((Hh, H)),
            _full((1, H)),
            _full((H, Hh)),
        ],
        out_specs=pl.BlockSpec((be, H), lambda i: (i, 0)),
        out_shape=jax.ShapeDtypeStruct((E, H), jnp.float32),
    )(edge_attr, sa, sb, We0, Wfe0[:Hh], Wfe0[Hh:], bfe0[None], We1)

    dst2d = dst.reshape(E // 128, 128)
    zeros = jnp.zeros((N // _NS, H), jnp.float32)
    partials = _make_sc_scatter(N, E, H)(tw, dst2d, zeros)

    out = pl.pallas_call(
        _node_body,
        grid=(N // bn,),
        in_specs=[
            pl.BlockSpec((_NC, bn, H), lambda i: (0, i, 0)),
            pl.BlockSpec((bn, Hh), lambda i: (i, 0)),
            _full((Hh, H)),
            _full((Hh, H)),
            _full((1, H)),
            _full((H, Hh)),
            _full((Hh, H)),
            _full((Hh, H)),
            _full((1, H)),
        ],
        out_specs=pl.BlockSpec((bn, H), lambda i: (i, 0)),
        out_shape=jax.ShapeDtypeStruct((N, H), jnp.float32),
    )(partials, h0, Wfn0[:Hh], Wfn0[Hh:], bfn0[None], Wn1,
      Wfn1[:Hh], Wfn1[Hh:], bfn1[None])

    return out


# speedup vs baseline: 4.0904x; 1.1222x over previous
import functools

import jax
import jax.numpy as jnp
from jax import lax
from jax.experimental import pallas as pl
from jax.experimental.pallas import tpu as pltpu
from jax.experimental.pallas import tpu_sc as plsc

_C = 1.0 / (1.0 + 1e-5) ** 0.5

_NC = 2
_NS = 16
_NW = _NC * _NS


def _h0p0_body(x_ref, wn0_ref, wfa_ref, h0_ref, p0_ref):
    h0 = jax.nn.relu(x_ref[...] @ wn0_ref[...])
    h0_ref[...] = h0
    p0_ref[...] = h0 @ wfa_ref[...]


def _edge_body(ea_ref, sa_ref, sb_ref, we0_ref, wfb_ref, bfe0_ref,
               we1_ref, tw_ref):
    t = jax.nn.relu(ea_ref[...] @ we0_ref[...])
    v = jax.nn.relu(sa_ref[...] + sb_ref[...] + t @ wfb_ref[...]
                    + bfe0_ref[...])
    w = jax.nn.relu(v @ we1_ref[...])
    tw_ref[...] = jnp.concatenate([t, w], axis=1)


def _node_body(ap_ref, h0_ref, wfn0a_ref, wfn0b_ref, bfn0_ref, wn1_ref,
               wfn1a_ref, wfn1b_ref, bfn1_ref, out_ref):
    aggr = ap_ref[0] + ap_ref[1]
    node1 = jax.nn.relu(aggr[:, :64] @ wfn0a_ref[...]
                        + h0_ref[...] @ wfn0b_ref[...] + bfn0_ref[...]) * _C
    h1 = jax.nn.relu(node1 @ wn1_ref[...])
    out_ref[...] = jax.nn.relu(aggr[:, 64:] @ wfn1a_ref[...]
                               + h1 @ wfn1b_ref[...] + bfn1_ref[...]) * _C


def _full(shape):
    return pl.BlockSpec(shape, lambda i: (0,) * len(shape))


def _make_sc_gather(N, E, H):
    R = E // 128
    SC = 8
    n_super = R // SC
    tail_rows = R - n_super * SC
    base_nj = n_super // _NW
    extra = n_super - base_nj * _NW

    mesh = plsc.VectorSubcoreMesh(core_axis_name="c", subcore_axis_name="s",
                                  num_cores=_NC, num_subcores=_NS)

    @functools.partial(
        pl.kernel,
        out_type=[jax.ShapeDtypeStruct((E, H), jnp.float32),
                  jax.ShapeDtypeStruct((E, H), jnp.float32)],
        mesh=mesh,
        scratch_types=[
            pltpu.VMEM((SC, 128), jnp.int32),
            pltpu.VMEM((SC, 128), jnp.int32),
            pltpu.VMEM((128, H), jnp.float32),
            pltpu.VMEM((128, H), jnp.float32),
            pltpu.SemaphoreType.DMA,
            pltpu.SemaphoreType.DMA,
        ],
    )
    def gather(p0_hbm, src2_hbm, dst2_hbm,
               sa_hbm, sb_hbm, idxa, idxb, rowsa, rowsb, sema, semb):
        w = lax.axis_index("s") * _NC + lax.axis_index("c")
        nj = jnp.where(w < extra, base_nj + 1, base_nj)

        def body(j, carry):
            r0 = (w + _NW * j) * SC
            pltpu.sync_copy(src2_hbm.at[pl.ds(r0, SC)], idxa)
            pltpu.sync_copy(dst2_hbm.at[pl.ds(r0, SC)], idxb)
            for t in range(SC):
                ca = pltpu.async_copy(p0_hbm.at[idxa.at[t]], rowsa, sema)
                cb = pltpu.async_copy(p0_hbm.at[idxb.at[t]], rowsb, semb)
                ca.wait()
                cb.wait()
                pltpu.sync_copy(rowsa, sa_hbm.at[pl.ds((r0 + t) * 128, 128)])
                pltpu.sync_copy(rowsb, sb_hbm.at[pl.ds((r0 + t) * 128, 128)])
            return carry

        lax.fori_loop(0, nj, body, 0)

        if tail_rows:
            @pl.when(w == 0)
            def _():
                r0 = n_super * SC
                pltpu.sync_copy(src2_hbm.at[pl.ds(r0, tail_rows)],
                                idxa.at[pl.ds(0, tail_rows)])
                pltpu.sync_copy(dst2_hbm.at[pl.ds(r0, tail_rows)],
                                idxb.at[pl.ds(0, tail_rows)])
                for t in range(tail_rows):
                    ca = pltpu.async_copy(p0_hbm.at[idxa.at[t]], rowsa, sema)
                    cb = pltpu.async_copy(p0_hbm.at[idxb.at[t]], rowsb, semb)
                    ca.wait()
                    cb.wait()
                    pltpu.sync_copy(rowsa,
                                    sa_hbm.at[pl.ds((r0 + t) * 128, 128)])
                    pltpu.sync_copy(rowsb,
                                    sb_hbm.at[pl.ds((r0 + t) * 128, 128)])

    return gather


def _make_sc_scatter(N, E, H):
    R = E // 128
    SC = 8
    n_super = R // SC
    tail_rows = R - n_super * SC
    base_nj = n_super // _NW
    extra = n_super - base_nj * _NW
    rps = (N // _NS) // 8 * 8
    last_rows = N - rps * (_NS - 1)

    mesh = plsc.VectorSubcoreMesh(core_axis_name="c", subcore_axis_name="s",
                                  num_cores=_NC, num_subcores=_NS)

    @functools.partial(
        pl.kernel,
        out_type=jax.ShapeDtypeStruct((_NC, N, H), jnp.float32),
        mesh=mesh,
        scratch_types=[
            pltpu.VMEM((SC, 128), jnp.int32),
            pltpu.VMEM((256, H), jnp.float32),
            pltpu.VMEM_SHARED((N, H), jnp.float32),
        ],
    )
    def scatter(tw_hbm, dst2_hbm, zeros_hbm, out_hbm, idxk, twbuf, acc):
        c = lax.axis_index("c")
        sid = lax.axis_index("s")
        w = sid * _NC + c

        @pl.when(sid < _NS - 1)
        def _():
            pltpu.sync_copy(zeros_hbm.at[pl.ds(0, rps)],
                            acc.at[pl.ds(sid * rps, rps)])

        @pl.when(sid == _NS - 1)
        def _():
            pltpu.sync_copy(zeros_hbm.at[pl.ds(0, last_rows)],
                            acc.at[pl.ds(sid * rps, last_rows)])

        plsc.subcore_barrier()

        nj = jnp.where(w < extra, base_nj + 1, base_nj)

        def body(j, carry):
            r0 = (w + _NW * j) * SC
            pltpu.sync_copy(dst2_hbm.at[pl.ds(r0, SC)], idxk)
            for t in range(SC // 2):
                pltpu.sync_copy(tw_hbm.at[pl.ds((r0 + 2 * t) * 128, 256)],
                                twbuf)
                pltpu.sync_copy(twbuf.at[pl.ds(0, 128)],
                                acc.at[idxk.at[2 * t]], add=True)
                pltpu.sync_copy(twbuf.at[pl.ds(128, 128)],
                                acc.at[idxk.at[2 * t + 1]], add=True)
            return carry

        lax.fori_loop(0, nj, body, 0)

        if tail_rows:
            @pl.when(w == 0)
            def _():
                r0 = n_super * SC
                pltpu.sync_copy(dst2_hbm.at[pl.ds(r0, tail_rows)],
                                idxk.at[pl.ds(0, tail_rows)])
                for t in range(tail_rows):
                    pltpu.sync_copy(tw_hbm.at[pl.ds((r0 + t) * 128, 128)],
                                    twbuf.at[pl.ds(0, 128)])
                    pltpu.sync_copy(twbuf.at[pl.ds(0, 128)],
                                    acc.at[idxk.at[t]], add=True)

        plsc.subcore_barrier()

        @pl.when(sid < _NS - 1)
        def _():
            pltpu.sync_copy(acc.at[pl.ds(sid * rps, rps)],
                            out_hbm.at[c, pl.ds(sid * rps, rps)])

        @pl.when(sid == _NS - 1)
        def _():
            pltpu.sync_copy(acc.at[pl.ds(sid * rps, last_rows)],
                            out_hbm.at[c, pl.ds(sid * rps, last_rows)])

    return scatter


def kernel(x, edge_index, edge_attr, Wn0, We0, Wfn0, bfn0, Wfe0, bfe0,
           Wn1, We1, Wfn1, bfn1, Wfe1, bfe1):
    N, DN = x.shape
    E, DE = edge_attr.shape
    H = Wfn0.shape[0]
    Hh = H // 2
    src = edge_index[0]
    dst = edge_index[1]
    src2 = src.reshape(E // 128, 128)
    dst2 = dst.reshape(E // 128, 128)

    bn = 2000 if N % 2000 == 0 else N
    be = 4000 if E % 4000 == 0 else E

    h0, p0 = pl.pallas_call(
        _h0p0_body,
        grid=(N // bn,),
        in_specs=[pl.BlockSpec((bn, DN), lambda i: (i, 0)), _full((DN, Hh)),
                  _full((Hh, H))],
        out_specs=[pl.BlockSpec((bn, Hh), lambda i: (i, 0)),
                   pl.BlockSpec((bn, H), lambda i: (i, 0))],
        out_shape=[jax.ShapeDtypeStruct((N, Hh), jnp.float32),
                   jax.ShapeDtypeStruct((N, H), jnp.float32)],
    )(x, Wn0, Wfe0[:Hh])

    sa, sb = _make_sc_gather(N, E, H)(p0, src2, dst2)

    tw = pl.pallas_call(
        _edge_body,
        grid=(E // be,),
        in_specs=[
            pl.BlockSpec((be, DE), lambda i: (i, 0)),
            pl.BlockSpec((be, H), lambda i: (i, 0)),
            pl.BlockSpec((be, H), lambda i: (i, 0)),
            _full((DE, Hh)),
            _full((Hh, H)),
            _full((1, H)),
            _full((H, Hh)),
        ],
        out_specs=pl.BlockSpec((be, H), lambda i: (i, 0)),
        out_shape=jax.ShapeDtypeStruct((E, H), jnp.float32),
    )(edge_attr, sa, sb, We0, Wfe0[Hh:], bfe0[None], We1)

    zeros = jnp.zeros((N - (N // _NS) // 8 * 8 * (_NS - 1), H), jnp.float32)
    partials = _make_sc_scatter(N, E, H)(tw, dst2, zeros)

    out = pl.pallas_call(
        _node_body,
        grid=(N // bn,),
        in_specs=[
            pl.BlockSpec((_NC, bn, H), lambda i: (0, i, 0)),
            pl.BlockSpec((bn, Hh), lambda i: (i, 0)),
            _full((Hh, H)),
            _full((Hh, H)),
            _full((1, H)),
            _full((H, Hh)),
            _full((Hh, H)),
            _full((Hh, H)),
            _full((1, H)),
        ],
        out_specs=pl.BlockSpec((bn, H), lambda i: (i, 0)),
        out_shape=jax.ShapeDtypeStruct((N, H), jnp.float32),
    )(partials, h0, Wfn0[:Hh], Wfn0[Hh:], bfn0[None], Wn1,
      Wfn1[:Hh], Wfn1[Hh:], bfn1[None])

    return out


# speedup vs baseline: 4.4273x; 1.0824x over previous
import functools

import jax
import jax.numpy as jnp
from jax import lax
from jax.experimental import pallas as pl
from jax.experimental.pallas import tpu as pltpu
from jax.experimental.pallas import tpu_sc as plsc

_C = 1.0 / (1.0 + 1e-5) ** 0.5

_NC = 2
_NS = 16
_NW = _NC * _NS


def _h0p0_body(x_ref, wn0_ref, wfa_ref, h0_ref, p0_ref):
    h0 = jax.nn.relu(x_ref[...] @ wn0_ref[...])
    h0_ref[...] = h0
    p0_ref[...] = h0 @ wfa_ref[...]


def _edge_body(ea_ref, sa_ref, sb_ref, we0_ref, wfb_ref, bfe0_ref,
               we1_ref, tw_ref):
    t = jax.nn.relu(ea_ref[...] @ we0_ref[...])
    v = jax.nn.relu(sa_ref[...] + sb_ref[...] + t @ wfb_ref[...]
                    + bfe0_ref[...])
    w = jax.nn.relu(v @ we1_ref[...])
    tw_ref[...] = jnp.concatenate([t, w], axis=1)


def _node_body(apa_ref, apb_ref, h0_ref, wfn0a_ref, wfn0b_ref, bfn0_ref,
               wn1_ref, wfn1a_ref, wfn1b_ref, bfn1_ref, out_ref):
    aggr = apa_ref[0] + apa_ref[1] + apb_ref[0] + apb_ref[1]
    node1 = jax.nn.relu(aggr[:, :64] @ wfn0a_ref[...]
                        + h0_ref[...] @ wfn0b_ref[...] + bfn0_ref[...]) * _C
    h1 = jax.nn.relu(node1 @ wn1_ref[...])
    out_ref[...] = jax.nn.relu(aggr[:, 64:] @ wfn1a_ref[...]
                               + h1 @ wfn1b_ref[...] + bfn1_ref[...]) * _C


def _full(shape):
    return pl.BlockSpec(shape, lambda i: (0,) * len(shape))


def _make_sc_gather(N, H, row0, n_rows):
    SC = 8
    n_super = n_rows // SC
    tail_rows = n_rows - n_super * SC
    base_nj = n_super // _NW
    extra = n_super - base_nj * _NW
    E_s = n_rows * 128

    mesh = plsc.VectorSubcoreMesh(core_axis_name="c", subcore_axis_name="s",
                                  num_cores=_NC, num_subcores=_NS)

    @functools.partial(
        pl.kernel,
        out_type=[jax.ShapeDtypeStruct((E_s, H), jnp.float32),
                  jax.ShapeDtypeStruct((E_s, H), jnp.float32)],
        mesh=mesh,
        scratch_types=[
            pltpu.VMEM((SC, 128), jnp.int32),
            pltpu.VMEM((SC, 128), jnp.int32),
            pltpu.VMEM((128, H), jnp.float32),
            pltpu.VMEM((128, H), jnp.float32),
            pltpu.SemaphoreType.DMA,
            pltpu.SemaphoreType.DMA,
        ],
    )
    def gather(p0_hbm, src2_hbm, dst2_hbm,
               sa_hbm, sb_hbm, idxa, idxb, rowsa, rowsb, sema, semb):
        w = lax.axis_index("s") * _NC + lax.axis_index("c")
        nj = jnp.where(w < extra, base_nj + 1, base_nj)

        def body(j, carry):
            r = (w + _NW * j) * SC
            pltpu.sync_copy(src2_hbm.at[pl.ds(row0 + r, SC)], idxa)
            pltpu.sync_copy(dst2_hbm.at[pl.ds(row0 + r, SC)], idxb)
            for t in range(SC):
                ca = pltpu.async_copy(p0_hbm.at[idxa.at[t]], rowsa, sema)
                cb = pltpu.async_copy(p0_hbm.at[idxb.at[t]], rowsb, semb)
                ca.wait()
                cb.wait()
                pltpu.sync_copy(rowsa, sa_hbm.at[pl.ds((r + t) * 128, 128)])
                pltpu.sync_copy(rowsb, sb_hbm.at[pl.ds((r + t) * 128, 128)])
            return carry

        lax.fori_loop(0, nj, body, 0)

        if tail_rows:
            @pl.when(w == 0)
            def _():
                r = n_super * SC
                pltpu.sync_copy(src2_hbm.at[pl.ds(row0 + r, tail_rows)],
                                idxa.at[pl.ds(0, tail_rows)])
                pltpu.sync_copy(dst2_hbm.at[pl.ds(row0 + r, tail_rows)],
                                idxb.at[pl.ds(0, tail_rows)])
                for t in range(tail_rows):
                    ca = pltpu.async_copy(p0_hbm.at[idxa.at[t]], rowsa, sema)
                    cb = pltpu.async_copy(p0_hbm.at[idxb.at[t]], rowsb, semb)
                    ca.wait()
                    cb.wait()
                    pltpu.sync_copy(rowsa,
                                    sa_hbm.at[pl.ds((r + t) * 128, 128)])
                    pltpu.sync_copy(rowsb,
                                    sb_hbm.at[pl.ds((r + t) * 128, 128)])

    return gather


def _make_sc_scatter(N, H, row0, n_rows):
    SC = 8
    n_super = n_rows // SC
    tail_rows = n_rows - n_super * SC
    base_nj = n_super // _NW
    extra = n_super - base_nj * _NW
    rps = (N // _NS) // 8 * 8
    last_rows = N - rps * (_NS - 1)

    mesh = plsc.VectorSubcoreMesh(core_axis_name="c", subcore_axis_name="s",
                                  num_cores=_NC, num_subcores=_NS)

    @functools.partial(
        pl.kernel,
        out_type=jax.ShapeDtypeStruct((_NC, N, H), jnp.float32),
        mesh=mesh,
        scratch_types=[
            pltpu.VMEM((SC, 128), jnp.int32),
            pltpu.VMEM((256, H), jnp.float32),
            pltpu.VMEM_SHARED((N, H), jnp.float32),
        ],
    )
    def scatter(tw_hbm, dst2_hbm, zeros_hbm, out_hbm, idxk, twbuf, acc):
        c = lax.axis_index("c")
        sid = lax.axis_index("s")
        w = sid * _NC + c

        @pl.when(sid < _NS - 1)
        def _():
            pltpu.sync_copy(zeros_hbm.at[pl.ds(0, rps)],
                            acc.at[pl.ds(sid * rps, rps)])

        @pl.when(sid == _NS - 1)
        def _():
            pltpu.sync_copy(zeros_hbm.at[pl.ds(0, last_rows)],
                            acc.at[pl.ds(sid * rps, last_rows)])

        plsc.subcore_barrier()

        nj = jnp.where(w < extra, base_nj + 1, base_nj)

        def body(j, carry):
            r = (w + _NW * j) * SC
            pltpu.sync_copy(dst2_hbm.at[pl.ds(row0 + r, SC)], idxk)
            for t in range(SC // 2):
                pltpu.sync_copy(tw_hbm.at[pl.ds((r + 2 * t) * 128, 256)],
                                twbuf)
                pltpu.sync_copy(twbuf.at[pl.ds(0, 128)],
                                acc.at[idxk.at[2 * t]], add=True)
                pltpu.sync_copy(twbuf.at[pl.ds(128, 128)],
                                acc.at[idxk.at[2 * t + 1]], add=True)
            return carry

        lax.fori_loop(0, nj, body, 0)

        if tail_rows:
            @pl.when(w == 0)
            def _():
                r = n_super * SC
                pltpu.sync_copy(dst2_hbm.at[pl.ds(row0 + r, tail_rows)],
                                idxk.at[pl.ds(0, tail_rows)])
                for t in range(tail_rows):
                    pltpu.sync_copy(tw_hbm.at[pl.ds((r + t) * 128, 128)],
                                    twbuf.at[pl.ds(0, 128)])
                    pltpu.sync_copy(twbuf.at[pl.ds(0, 128)],
                                    acc.at[idxk.at[t]], add=True)

        plsc.subcore_barrier()

        @pl.when(sid < _NS - 1)
        def _():
            pltpu.sync_copy(acc.at[pl.ds(sid * rps, rps)],
                            out_hbm.at[c, pl.ds(sid * rps, rps)])

        @pl.when(sid == _NS - 1)
        def _():
            pltpu.sync_copy(acc.at[pl.ds(sid * rps, last_rows)],
                            out_hbm.at[c, pl.ds(sid * rps, last_rows)])

    return scatter


def _edge_call(ea, sa, sb, We0, Wfb, bfe0, We1, e_base, E_s, be, DE, Hh, H):
    nb = e_base // be if e_base % be == 0 else None
    steps = E_s // be
    if nb is None:
        raise ValueError("slice base must be a multiple of the block size")
    return pl.pallas_call(
        _edge_body,
        grid=(steps,),
        in_specs=[
            pl.BlockSpec((be, DE), lambda i: (i + nb, 0)),
            pl.BlockSpec((be, H), lambda i: (i, 0)),
            pl.BlockSpec((be, H), lambda i: (i, 0)),
            _full((DE, Hh)),
            _full((Hh, H)),
            _full((1, H)),
            _full((H, Hh)),
        ],
        out_specs=pl.BlockSpec((be, H), lambda i: (i, 0)),
        out_shape=jax.ShapeDtypeStruct((E_s, H), jnp.float32),
    )(ea, sa, sb, We0, Wfb, bfe0, We1)


def kernel(x, edge_index, edge_attr, Wn0, We0, Wfn0, bfn0, Wfe0, bfe0,
           Wn1, We1, Wfn1, bfn1, Wfe1, bfe1):
    N, DN = x.shape
    E, DE = edge_attr.shape
    H = Wfn0.shape[0]
    Hh = H // 2
    src2 = edge_index[0].reshape(E // 128, 128)
    dst2 = edge_index[1].reshape(E // 128, 128)

    bn = 2000 if N % 2000 == 0 else N
    R = E // 128

    if E == 320000:
        rows_a = 1280
        be_a, be_b = 4096, 2560
    else:
        rows_a = (R // 16) * 8
        be_a = be_b = 128
    rows_b = R - rows_a
    E_a, E_b = rows_a * 128, rows_b * 128

    h0, p0 = pl.pallas_call(
        _h0p0_body,
        grid=(N // bn,),
        in_specs=[pl.BlockSpec((bn, DN), lambda i: (i, 0)), _full((DN, Hh)),
                  _full((Hh, H))],
        out_specs=[pl.BlockSpec((bn, Hh), lambda i: (i, 0)),
                   pl.BlockSpec((bn, H), lambda i: (i, 0))],
        out_shape=[jax.ShapeDtypeStruct((N, Hh), jnp.float32),
                   jax.ShapeDtypeStruct((N, H), jnp.float32)],
    )(x, Wn0, Wfe0[:Hh])

    sa_a, sb_a = _make_sc_gather(N, H, 0, rows_a)(p0, src2, dst2)
    sa_b, sb_b = _make_sc_gather(N, H, rows_a, rows_b)(p0, src2, dst2)

    zeros = jnp.zeros((N - (N // _NS) // 8 * 8 * (_NS - 1), H), jnp.float32)

    tw_a = _edge_call(edge_attr, sa_a, sb_a, We0, Wfe0[Hh:], bfe0[None], We1,
                      0, E_a, be_a, DE, Hh, H)
    part_a = _make_sc_scatter(N, H, 0, rows_a)(tw_a, dst2, zeros)

    tw_b = _edge_call(edge_attr, sa_b, sb_b, We0, Wfe0[Hh:], bfe0[None], We1,
                      E_a, E_b, be_b, DE, Hh, H)
    part_b = _make_sc_scatter(N, H, rows_a, rows_b)(tw_b, dst2, zeros)

    out = pl.pallas_call(
        _node_body,
        grid=(N // bn,),
        in_specs=[
            pl.BlockSpec((_NC, bn, H), lambda i: (0, i, 0)),
            pl.BlockSpec((_NC, bn, H), lambda i: (0, i, 0)),
            pl.BlockSpec((bn, Hh), lambda i: (i, 0)),
            _full((Hh, H)),
            _full((Hh, H)),
            _full((1, H)),
            _full((H, Hh)),
            _full((Hh, H)),
            _full((Hh, H)),
            _full((1, H)),
        ],
        out_specs=pl.BlockSpec((bn, H), lambda i: (i, 0)),
        out_shape=jax.ShapeDtypeStruct((N, H), jnp.float32),
    )(part_a, part_b, h0, Wfn0[:Hh], Wfn0[Hh:], bfn0[None], Wn1,
      Wfn1[:Hh], Wfn1[Hh:], bfn1[None])

    return out


# speedup vs baseline: 4.6125x; 1.0418x over previous
import functools

import jax
import jax.numpy as jnp
from jax import lax
from jax.experimental import pallas as pl
from jax.experimental.pallas import tpu as pltpu
from jax.experimental.pallas import tpu_sc as plsc

_C = 1.0 / (1.0 + 1e-5) ** 0.5

_NC = 2
_NS = 16
_NW = _NC * _NS


def _h0p0_body(x_ref, wn0_ref, wfa_ref, h0_ref, p0_ref):
    h0 = jax.nn.relu(x_ref[...] @ wn0_ref[...])
    h0_ref[...] = h0
    p0_ref[...] = h0 @ wfa_ref[...]


def _edge_body(ea_ref, sa_ref, sb_ref, we0_ref, wfb_ref, bfe0_ref,
               we1_ref, tw_ref):
    t = jax.nn.relu(ea_ref[...] @ we0_ref[...])
    v = jax.nn.relu(sa_ref[...] + sb_ref[...] + t @ wfb_ref[...]
                    + bfe0_ref[...])
    w = jax.nn.relu(v @ we1_ref[...])
    tw_ref[...] = jnp.concatenate([t, w], axis=1)


def _node_body(apa_ref, apb_ref, h0_ref, wfn0a_ref, wfn0b_ref, bfn0_ref,
               wn1_ref, wfn1a_ref, wfn1b_ref, bfn1_ref, out_ref):
    aggr = apa_ref[0] + apa_ref[1] + apb_ref[0] + apb_ref[1]
    node1 = jax.nn.relu(aggr[:, :64] @ wfn0a_ref[...]
                        + h0_ref[...] @ wfn0b_ref[...] + bfn0_ref[...]) * _C
    h1 = jax.nn.relu(node1 @ wn1_ref[...])
    out_ref[...] = jax.nn.relu(aggr[:, 64:] @ wfn1a_ref[...]
                               + h1 @ wfn1b_ref[...] + bfn1_ref[...]) * _C


def _full(shape):
    return pl.BlockSpec(shape, lambda i: (0,) * len(shape))


def _make_sc_gather(N, H, row0, n_rows):
    SC = 8
    n_super = n_rows // SC
    tail_rows = n_rows - n_super * SC
    base_nj = n_super // _NW
    extra = n_super - base_nj * _NW
    E_s = n_rows * 128

    mesh = plsc.VectorSubcoreMesh(core_axis_name="c", subcore_axis_name="s",
                                  num_cores=_NC, num_subcores=_NS)

    @functools.partial(
        pl.kernel,
        out_type=[jax.ShapeDtypeStruct((E_s, H), jnp.float32),
                  jax.ShapeDtypeStruct((E_s, H), jnp.float32)],
        mesh=mesh,
        scratch_types=[
            pltpu.VMEM((SC, 128), jnp.int32),
            pltpu.VMEM((SC, 128), jnp.int32),
            pltpu.VMEM((2, 128, H), jnp.float32),
            pltpu.VMEM((2, 128, H), jnp.float32),
            pltpu.SemaphoreType.DMA,
            pltpu.SemaphoreType.DMA,
        ],
    )
    def gather(p0_hbm, src2_hbm, dst2_hbm,
               sa_hbm, sb_hbm, idxa, idxb, rowsa, rowsb, semg, semw):
        w = lax.axis_index("s") * _NC + lax.axis_index("c")
        nj = jnp.where(w < extra, base_nj + 1, base_nj)

        def body(j, carry):
            r = (w + _NW * j) * SC
            pltpu.sync_copy(src2_hbm.at[pl.ds(row0 + r, SC)], idxa)
            pltpu.sync_copy(dst2_hbm.at[pl.ds(row0 + r, SC)], idxb)
            g = [None] * SC
            wr = [None] * SC

            def fire(t):
                s = t % 2
                g[t] = (pltpu.async_copy(p0_hbm.at[idxa.at[t]],
                                         rowsa.at[s], semg),
                        pltpu.async_copy(p0_hbm.at[idxb.at[t]],
                                         rowsb.at[s], semg))

            fire(0)
            for t in range(SC):
                s = t % 2
                if t >= 1:
                    for d in wr[t - 1]:
                        d.wait()
                if t + 1 < SC:
                    fire(t + 1)
                for d in g[t]:
                    d.wait()
                wr[t] = (
                    pltpu.async_copy(rowsa.at[s],
                                     sa_hbm.at[pl.ds((r + t) * 128, 128)],
                                     semw),
                    pltpu.async_copy(rowsb.at[s],
                                     sb_hbm.at[pl.ds((r + t) * 128, 128)],
                                     semw),
                )
            for d in wr[SC - 1]:
                d.wait()
            return carry

        lax.fori_loop(0, nj, body, 0)

        if tail_rows:
            @pl.when(w == 0)
            def _():
                r = n_super * SC
                pltpu.sync_copy(src2_hbm.at[pl.ds(row0 + r, tail_rows)],
                                idxa.at[pl.ds(0, tail_rows)])
                pltpu.sync_copy(dst2_hbm.at[pl.ds(row0 + r, tail_rows)],
                                idxb.at[pl.ds(0, tail_rows)])
                for t in range(tail_rows):
                    ca = pltpu.async_copy(p0_hbm.at[idxa.at[t]],
                                          rowsa.at[0], semg)
                    cb = pltpu.async_copy(p0_hbm.at[idxb.at[t]],
                                          rowsb.at[0], semg)
                    ca.wait()
                    cb.wait()
                    pltpu.sync_copy(rowsa.at[0],
                                    sa_hbm.at[pl.ds((r + t) * 128, 128)])
                    pltpu.sync_copy(rowsb.at[0],
                                    sb_hbm.at[pl.ds((r + t) * 128, 128)])

    return gather


def _make_sc_scatter(N, H, row0, n_rows):
    SC = 8
    n_super = n_rows // SC
    tail_rows = n_rows - n_super * SC
    base_nj = n_super // _NW
    extra = n_super - base_nj * _NW
    rps = (N // _NS) // 8 * 8
    last_rows = N - rps * (_NS - 1)

    mesh = plsc.VectorSubcoreMesh(core_axis_name="c", subcore_axis_name="s",
                                  num_cores=_NC, num_subcores=_NS)

    @functools.partial(
        pl.kernel,
        out_type=jax.ShapeDtypeStruct((_NC, N, H), jnp.float32),
        mesh=mesh,
        scratch_types=[
            pltpu.VMEM((SC, 128), jnp.int32),
            pltpu.VMEM((2, 128, H), jnp.float32),
            pltpu.VMEM_SHARED((N, H), jnp.float32),
            pltpu.SemaphoreType.DMA,
            pltpu.SemaphoreType.DMA,
        ],
    )
    def scatter(tw_hbm, dst2_hbm, zeros_hbm, out_hbm, idxk, twbuf, acc,
                seml, sems):
        c = lax.axis_index("c")
        sid = lax.axis_index("s")
        w = sid * _NC + c

        @pl.when(sid < _NS - 1)
        def _():
            pltpu.sync_copy(zeros_hbm.at[pl.ds(0, rps)],
                            acc.at[pl.ds(sid * rps, rps)])

        @pl.when(sid == _NS - 1)
        def _():
            pltpu.sync_copy(zeros_hbm.at[pl.ds(0, last_rows)],
                            acc.at[pl.ds(sid * rps, last_rows)])

        plsc.subcore_barrier()

        nj = jnp.where(w < extra, base_nj + 1, base_nj)

        def body(j, carry):
            r = (w + _NW * j) * SC
            pltpu.sync_copy(dst2_hbm.at[pl.ds(row0 + r, SC)], idxk)
            ld = [None] * SC
            sc_ = [None] * SC

            def fire(t):
                ld[t] = pltpu.async_copy(
                    tw_hbm.at[pl.ds((r + t) * 128, 128)],
                    twbuf.at[t % 2], seml)

            fire(0)
            for t in range(SC):
                s = t % 2
                if t >= 1:
                    sc_[t - 1].wait()
                if t + 1 < SC:
                    fire(t + 1)
                ld[t].wait()
                sc_[t] = pltpu.async_copy(twbuf.at[s],
                                          acc.at[idxk.at[t]], sems,
                                          add=True)
            sc_[SC - 1].wait()
            return carry

        lax.fori_loop(0, nj, body, 0)

        if tail_rows:
            @pl.when(w == 0)
            def _():
                r = n_super * SC
                pltpu.sync_copy(dst2_hbm.at[pl.ds(row0 + r, tail_rows)],
                                idxk.at[pl.ds(0, tail_rows)])
                for t in range(tail_rows):
                    pltpu.sync_copy(tw_hbm.at[pl.ds((r + t) * 128, 128)],
                                    twbuf.at[0])
                    pltpu.sync_copy(twbuf.at[0],
                                    acc.at[idxk.at[t]], add=True)

        plsc.subcore_barrier()

        @pl.when(sid < _NS - 1)
        def _():
            pltpu.sync_copy(acc.at[pl.ds(sid * rps, rps)],
                            out_hbm.at[c, pl.ds(sid * rps, rps)])

        @pl.when(sid == _NS - 1)
        def _():
            pltpu.sync_copy(acc.at[pl.ds(sid * rps, last_rows)],
                            out_hbm.at[c, pl.ds(sid * rps, last_rows)])

    return scatter


def _edge_call(ea, sa, sb, We0, Wfb, bfe0, We1, e_base, E_s, be, DE, Hh, H):
    nb = e_base // be if e_base % be == 0 else None
    steps = E_s // be
    if nb is None:
        raise ValueError("slice base must be a multiple of the block size")
    return pl.pallas_call(
        _edge_body,
        grid=(steps,),
        in_specs=[
            pl.BlockSpec((be, DE), lambda i: (i + nb, 0)),
            pl.BlockSpec((be, H), lambda i: (i, 0)),
            pl.BlockSpec((be, H), lambda i: (i, 0)),
            _full((DE, Hh)),
            _full((Hh, H)),
            _full((1, H)),
            _full((H, Hh)),
        ],
        out_specs=pl.BlockSpec((be, H), lambda i: (i, 0)),
        out_shape=jax.ShapeDtypeStruct((E_s, H), jnp.float32),
    )(ea, sa, sb, We0, Wfb, bfe0, We1)


def kernel(x, edge_index, edge_attr, Wn0, We0, Wfn0, bfn0, Wfe0, bfe0,
           Wn1, We1, Wfn1, bfn1, Wfe1, bfe1):
    N, DN = x.shape
    E, DE = edge_attr.shape
    H = Wfn0.shape[0]
    Hh = H // 2
    src2 = edge_index[0].reshape(E // 128, 128)
    dst2 = edge_index[1].reshape(E // 128, 128)

    bn = 2000 if N % 2000 == 0 else N
    R = E // 128

    if E == 320000:
        rows_a = 1280
        be_a, be_b = 4096, 2560
    else:
        rows_a = (R // 16) * 8
        be_a = be_b = 128
    rows_b = R - rows_a
    E_a, E_b = rows_a * 128, rows_b * 128

    h0, p0 = pl.pallas_call(
        _h0p0_body,
        grid=(N // bn,),
        in_specs=[pl.BlockSpec((bn, DN), lambda i: (i, 0)), _full((DN, Hh)),
                  _full((Hh, H))],
        out_specs=[pl.BlockSpec((bn, Hh), lambda i: (i, 0)),
                   pl.BlockSpec((bn, H), lambda i: (i, 0))],
        out_shape=[jax.ShapeDtypeStruct((N, Hh), jnp.float32),
                   jax.ShapeDtypeStruct((N, H), jnp.float32)],
    )(x, Wn0, Wfe0[:Hh])

    sa_a, sb_a = _make_sc_gather(N, H, 0, rows_a)(p0, src2, dst2)
    sa_b, sb_b = _make_sc_gather(N, H, rows_a, rows_b)(p0, src2, dst2)

    zeros = jnp.zeros((N - (N // _NS) // 8 * 8 * (_NS - 1), H), jnp.float32)

    tw_a = _edge_call(edge_attr, sa_a, sb_a, We0, Wfe0[Hh:], bfe0[None], We1,
                      0, E_a, be_a, DE, Hh, H)
    part_a = _make_sc_scatter(N, H, 0, rows_a)(tw_a, dst2, zeros)

    tw_b = _edge_call(edge_attr, sa_b, sb_b, We0, Wfe0[Hh:], bfe0[None], We1,
                      E_a, E_b, be_b, DE, Hh, H)
    part_b = _make_sc_scatter(N, H, rows_a, rows_b)(tw_b, dst2, zeros)

    out = pl.pallas_call(
        _node_body,
        grid=(N // bn,),
        in_specs=[
            pl.BlockSpec((_NC, bn, H), lambda i: (0, i, 0)),
            pl.BlockSpec((_NC, bn, H), lambda i: (0, i, 0)),
            pl.BlockSpec((bn, Hh), lambda i: (i, 0)),
            _full((Hh, H)),
            _full((Hh, H)),
            _full((1, H)),
            _full((H, Hh)),
            _full((Hh, H)),
            _full((Hh, H)),
            _full((1, H)),
        ],
        out_specs=pl.BlockSpec((bn, H), lambda i: (i, 0)),
        out_shape=jax.ShapeDtypeStruct((N, H), jnp.float32),
    )(part_a, part_b, h0, Wfn0[:Hh], Wfn0[Hh:], bfn0[None], Wn1,
      Wfn1[:Hh], Wfn1[Hh:], bfn1[None])

    return out


# speedup vs baseline: 5.5455x; 1.2023x over previous
import functools

import jax
import jax.numpy as jnp
from jax import lax
from jax.experimental import pallas as pl
from jax.experimental.pallas import tpu as pltpu
from jax.experimental.pallas import tpu_sc as plsc

_C = 1.0 / (1.0 + 1e-5) ** 0.5

_NC = 2
_NS = 16
_NW = _NC * _NS


def _h0p0_body(x_ref, wn0_ref, wfa_ref, h0_ref, p0_ref):
    h0 = jax.nn.relu(x_ref[...] @ wn0_ref[...])
    h0_ref[...] = h0
    p0_ref[...] = h0 @ wfa_ref[...]


def _edge_body(ea_ref, sa_ref, sb_ref, we0_ref, wfb_ref, bfe0_ref,
               we1_ref, tw_ref):
    t = jax.nn.relu(ea_ref[...] @ we0_ref[...])
    v = jax.nn.relu(sa_ref[...] + sb_ref[...] + t @ wfb_ref[...]
                    + bfe0_ref[...])
    w = jax.nn.relu(v @ we1_ref[...])
    tw_ref[...] = jnp.concatenate([t, w], axis=1)


def _node_body(apa_ref, apb_ref, h0_ref, wfn0a_ref, wfn0b_ref, bfn0_ref,
               wn1_ref, wfn1a_ref, wfn1b_ref, bfn1_ref, out_ref):
    aggr = apa_ref[0] + apa_ref[1] + apb_ref[0] + apb_ref[1]
    node1 = jax.nn.relu(aggr[:, :64] @ wfn0a_ref[...]
                        + h0_ref[...] @ wfn0b_ref[...] + bfn0_ref[...]) * _C
    h1 = jax.nn.relu(node1 @ wn1_ref[...])
    out_ref[...] = jax.nn.relu(aggr[:, 64:] @ wfn1a_ref[...]
                               + h1 @ wfn1b_ref[...] + bfn1_ref[...]) * _C


def _full(shape):
    return pl.BlockSpec(shape, lambda i: (0,) * len(shape))


def _make_sc_gather(N, H, row0, n_rows):
    SC = 8
    n_super = n_rows // SC
    tail_rows = n_rows - n_super * SC
    base_nj = n_super // _NW
    extra = n_super - base_nj * _NW
    E_s = n_rows * 128
    rps = (N // _NS) // 8 * 8
    last_rows = N - rps * (_NS - 1)

    mesh = plsc.VectorSubcoreMesh(core_axis_name="c", subcore_axis_name="s",
                                  num_cores=_NC, num_subcores=_NS)

    @functools.partial(
        pl.kernel,
        out_type=[jax.ShapeDtypeStruct((E_s, H), jnp.float32),
                  jax.ShapeDtypeStruct((E_s, H), jnp.float32)],
        mesh=mesh,
        scratch_types=[
            pltpu.VMEM((SC, 128), jnp.int32),
            pltpu.VMEM((SC, 128), jnp.int32),
            pltpu.VMEM((128, H), jnp.float32),
            pltpu.VMEM((128, H), jnp.float32),
            pltpu.VMEM_SHARED((N, H), jnp.float32),
            pltpu.SemaphoreType.DMA,
            pltpu.SemaphoreType.DMA,
        ],
    )
    def gather(p0_hbm, src2_hbm, dst2_hbm,
               sa_hbm, sb_hbm, idxa, idxb, rowsa, rowsb, tbl, semg, semw):
        sid = lax.axis_index("s")
        w = sid * _NC + lax.axis_index("c")
        nj = jnp.where(w < extra, base_nj + 1, base_nj)

        @pl.when(sid < _NS - 1)
        def _():
            pltpu.sync_copy(p0_hbm.at[pl.ds(sid * rps, rps)],
                            tbl.at[pl.ds(sid * rps, rps)])

        @pl.when(sid == _NS - 1)
        def _():
            pltpu.sync_copy(p0_hbm.at[pl.ds(sid * rps, last_rows)],
                            tbl.at[pl.ds(sid * rps, last_rows)])

        plsc.subcore_barrier()

        def body(j, carry):
            r = (w + _NW * j) * SC
            pltpu.sync_copy(src2_hbm.at[pl.ds(row0 + r, SC)], idxa)
            pltpu.sync_copy(dst2_hbm.at[pl.ds(row0 + r, SC)], idxb)
            wa = [None] * SC
            wb = [None] * SC
            for t in range(SC):
                if t >= 1:
                    wa[t - 1].wait()
                pltpu.async_copy(tbl.at[idxa.at[t]], rowsa, semg).wait()
                wa[t] = pltpu.async_copy(
                    rowsa, sa_hbm.at[pl.ds((r + t) * 128, 128)], semw)
                if t >= 1:
                    wb[t - 1].wait()
                pltpu.async_copy(tbl.at[idxb.at[t]], rowsb, semg).wait()
                wb[t] = pltpu.async_copy(
                    rowsb, sb_hbm.at[pl.ds((r + t) * 128, 128)], semw)
            wa[SC - 1].wait()
            wb[SC - 1].wait()
            return carry

        lax.fori_loop(0, nj, body, 0)

        if tail_rows:
            @pl.when(w == 0)
            def _():
                r = n_super * SC
                pltpu.sync_copy(src2_hbm.at[pl.ds(row0 + r, tail_rows)],
                                idxa.at[pl.ds(0, tail_rows)])
                pltpu.sync_copy(dst2_hbm.at[pl.ds(row0 + r, tail_rows)],
                                idxb.at[pl.ds(0, tail_rows)])
                for t in range(tail_rows):
                    ca = pltpu.async_copy(tbl.at[idxa.at[t]], rowsa, semg)
                    cb = pltpu.async_copy(tbl.at[idxb.at[t]], rowsb, semg)
                    ca.wait()
                    cb.wait()
                    pltpu.sync_copy(rowsa,
                                    sa_hbm.at[pl.ds((r + t) * 128, 128)])
                    pltpu.sync_copy(rowsb,
                                    sb_hbm.at[pl.ds((r + t) * 128, 128)])

    return gather


def _make_sc_scatter(N, H, row0, n_rows):
    SC = 8
    n_super = n_rows // SC
    tail_rows = n_rows - n_super * SC
    base_nj = n_super // _NW
    extra = n_super - base_nj * _NW
    rps = (N // _NS) // 8 * 8
    last_rows = N - rps * (_NS - 1)

    mesh = plsc.VectorSubcoreMesh(core_axis_name="c", subcore_axis_name="s",
                                  num_cores=_NC, num_subcores=_NS)

    @functools.partial(
        pl.kernel,
        out_type=jax.ShapeDtypeStruct((_NC, N, H), jnp.float32),
        mesh=mesh,
        scratch_types=[
            pltpu.VMEM((SC, 128), jnp.int32),
            pltpu.VMEM((2, 128, H), jnp.float32),
            pltpu.VMEM_SHARED((N, H), jnp.float32),
            pltpu.SemaphoreType.DMA,
            pltpu.SemaphoreType.DMA,
        ],
    )
    def scatter(tw_hbm, dst2_hbm, zeros_hbm, out_hbm, idxk, twbuf, acc,
                seml, sems):
        c = lax.axis_index("c")
        sid = lax.axis_index("s")
        w = sid * _NC + c

        @pl.when(sid < _NS - 1)
        def _():
            pltpu.sync_copy(zeros_hbm.at[pl.ds(0, rps)],
                            acc.at[pl.ds(sid * rps, rps)])

        @pl.when(sid == _NS - 1)
        def _():
            pltpu.sync_copy(zeros_hbm.at[pl.ds(0, last_rows)],
                            acc.at[pl.ds(sid * rps, last_rows)])

        plsc.subcore_barrier()

        nj = jnp.where(w < extra, base_nj + 1, base_nj)

        def body(j, carry):
            r = (w + _NW * j) * SC
            pltpu.sync_copy(dst2_hbm.at[pl.ds(row0 + r, SC)], idxk)
            ld = [None] * SC
            sc_ = [None] * SC

            def fire(t):
                ld[t] = pltpu.async_copy(
                    tw_hbm.at[pl.ds((r + t) * 128, 128)],
                    twbuf.at[t % 2], seml)

            fire(0)
            for t in range(SC):
                s = t % 2
                if t >= 1:
                    sc_[t - 1].wait()
                if t + 1 < SC:
                    fire(t + 1)
                ld[t].wait()
                sc_[t] = pltpu.async_copy(twbuf.at[s],
                                          acc.at[idxk.at[t]], sems,
                                          add=True)
            sc_[SC - 1].wait()
            return carry

        lax.fori_loop(0, nj, body, 0)

        if tail_rows:
            @pl.when(w == 0)
            def _():
                r = n_super * SC
                pltpu.sync_copy(dst2_hbm.at[pl.ds(row0 + r, tail_rows)],
                                idxk.at[pl.ds(0, tail_rows)])
                for t in range(tail_rows):
                    pltpu.sync_copy(tw_hbm.at[pl.ds((r + t) * 128, 128)],
                                    twbuf.at[0])
                    pltpu.sync_copy(twbuf.at[0],
                                    acc.at[idxk.at[t]], add=True)

        plsc.subcore_barrier()

        @pl.when(sid < _NS - 1)
        def _():
            pltpu.sync_copy(acc.at[pl.ds(sid * rps, rps)],
                            out_hbm.at[c, pl.ds(sid * rps, rps)])

        @pl.when(sid == _NS - 1)
        def _():
            pltpu.sync_copy(acc.at[pl.ds(sid * rps, last_rows)],
                            out_hbm.at[c, pl.ds(sid * rps, last_rows)])

    return scatter


def _edge_call(ea, sa, sb, We0, Wfb, bfe0, We1, e_base, E_s, be, DE, Hh, H):
    nb = e_base // be if e_base % be == 0 else None
    steps = E_s // be
    if nb is None:
        raise ValueError("slice base must be a multiple of the block size")
    return pl.pallas_call(
        _edge_body,
        grid=(steps,),
        in_specs=[
            pl.BlockSpec((be, DE), lambda i: (i + nb, 0)),
            pl.BlockSpec((be, H), lambda i: (i, 0)),
            pl.BlockSpec((be, H), lambda i: (i, 0)),
            _full((DE, Hh)),
            _full((Hh, H)),
            _full((1, H)),
            _full((H, Hh)),
        ],
        out_specs=pl.BlockSpec((be, H), lambda i: (i, 0)),
        out_shape=jax.ShapeDtypeStruct((E_s, H), jnp.float32),
    )(ea, sa, sb, We0, Wfb, bfe0, We1)


def kernel(x, edge_index, edge_attr, Wn0, We0, Wfn0, bfn0, Wfe0, bfe0,
           Wn1, We1, Wfn1, bfn1, Wfe1, bfe1):
    N, DN = x.shape
    E, DE = edge_attr.shape
    H = Wfn0.shape[0]
    Hh = H // 2
    src2 = edge_index[0].reshape(E // 128, 128)
    dst2 = edge_index[1].reshape(E // 128, 128)

    bn = 2000 if N % 2000 == 0 else N
    R = E // 128

    if E == 320000:
        rows_a = 1280
        be_a, be_b = 4096, 2560
    else:
        rows_a = (R // 16) * 8
        be_a = be_b = 128
    rows_b = R - rows_a
    E_a, E_b = rows_a * 128, rows_b * 128

    h0, p0 = pl.pallas_call(
        _h0p0_body,
        grid=(N // bn,),
        in_specs=[pl.BlockSpec((bn, DN), lambda i: (i, 0)), _full((DN, Hh)),
                  _full((Hh, H))],
        out_specs=[pl.BlockSpec((bn, Hh), lambda i: (i, 0)),
                   pl.BlockSpec((bn, H), lambda i: (i, 0))],
        out_shape=[jax.ShapeDtypeStruct((N, Hh), jnp.float32),
                   jax.ShapeDtypeStruct((N, H), jnp.float32)],
    )(x, Wn0, Wfe0[:Hh])

    sa_a, sb_a = _make_sc_gather(N, H, 0, rows_a)(p0, src2, dst2)
    sa_b, sb_b = _make_sc_gather(N, H, rows_a, rows_b)(p0, src2, dst2)

    zeros = jnp.zeros((N - (N // _NS) // 8 * 8 * (_NS - 1), H), jnp.float32)

    tw_a = _edge_call(edge_attr, sa_a, sb_a, We0, Wfe0[Hh:], bfe0[None], We1,
                      0, E_a, be_a, DE, Hh, H)
    part_a = _make_sc_scatter(N, H, 0, rows_a)(tw_a, dst2, zeros)

    tw_b = _edge_call(edge_attr, sa_b, sb_b, We0, Wfe0[Hh:], bfe0[None], We1,
                      E_a, E_b, be_b, DE, Hh, H)
    part_b = _make_sc_scatter(N, H, rows_a, rows_b)(tw_b, dst2, zeros)

    out = pl.pallas_call(
        _node_body,
        grid=(N // bn,),
        in_specs=[
            pl.BlockSpec((_NC, bn, H), lambda i: (0, i, 0)),
            pl.BlockSpec((_NC, bn, H), lambda i: (0, i, 0)),
            pl.BlockSpec((bn, Hh), lambda i: (i, 0)),
            _full((Hh, H)),
            _full((Hh, H)),
            _full((1, H)),
            _full((H, Hh)),
            _full((Hh, H)),
            _full((Hh, H)),
            _full((1, H)),
        ],
        out_specs=pl.BlockSpec((bn, H), lambda i: (i, 0)),
        out_shape=jax.ShapeDtypeStruct((N, H), jnp.float32),
    )(part_a, part_b, h0, Wfn0[:Hh], Wfn0[Hh:], bfn0[None], Wn1,
      Wfn1[:Hh], Wfn1[Hh:], bfn1[None])

    return out


# speedup vs baseline: 6.7115x; 1.2103x over previous
import functools

import jax
import jax.numpy as jnp
from jax import lax
from jax.experimental import pallas as pl
from jax.experimental.pallas import tpu as pltpu
from jax.experimental.pallas import tpu_sc as plsc

_C = 1.0 / (1.0 + 1e-5) ** 0.5

_NC = 2
_NS = 16
_NW = _NC * _NS


def _h0p0_body(x_ref, wn0_ref, wfa_ref, h0_ref, p0_ref):
    h0 = jax.nn.relu(x_ref[...] @ wn0_ref[...])
    h0_ref[...] = h0
    p0_ref[...] = h0 @ wfa_ref[...]


def _edge_body(ea_ref, s_ref, we0_ref, wfb_ref, bfe0_ref,
               we1_ref, tw_ref):
    t = jax.nn.relu(ea_ref[...] @ we0_ref[...])
    v = jax.nn.relu(s_ref[...] + t @ wfb_ref[...]
                    + bfe0_ref[...])
    w = jax.nn.relu(v @ we1_ref[...])
    tw_ref[...] = jnp.concatenate([t, w], axis=1)


def _node_body(apa_ref, apb_ref, h0_ref, wfn0a_ref, wfn0b_ref, bfn0_ref,
               wn1_ref, wfn1a_ref, wfn1b_ref, bfn1_ref, out_ref):
    aggr = apa_ref[0] + apa_ref[1] + apb_ref[0] + apb_ref[1]
    node1 = jax.nn.relu(aggr[:, :64] @ wfn0a_ref[...]
                        + h0_ref[...] @ wfn0b_ref[...] + bfn0_ref[...]) * _C
    h1 = jax.nn.relu(node1 @ wn1_ref[...])
    out_ref[...] = jax.nn.relu(aggr[:, 64:] @ wfn1a_ref[...]
                               + h1 @ wfn1b_ref[...] + bfn1_ref[...]) * _C


def _full(shape):
    return pl.BlockSpec(shape, lambda i: (0,) * len(shape))


def _make_sc_gather(N, H, row0, n_rows):
    SC = 8
    n_super = n_rows // SC
    tail_rows = n_rows - n_super * SC
    base_nj = n_super // _NW
    extra = n_super - base_nj * _NW
    E_s = n_rows * 128
    rps = (N // _NS) // 8 * 8
    last_rows = N - rps * (_NS - 1)

    mesh = plsc.VectorSubcoreMesh(core_axis_name="c", subcore_axis_name="s",
                                  num_cores=_NC, num_subcores=_NS)

    @functools.partial(
        pl.kernel,
        out_type=jax.ShapeDtypeStruct((E_s, H), jnp.float32),
        mesh=mesh,
        scratch_types=[
            pltpu.VMEM((SC, 128), jnp.int32),
            pltpu.VMEM((SC, 128), jnp.int32),
            pltpu.VMEM((2, 128, H), jnp.float32),
            pltpu.VMEM_SHARED((N, H), jnp.float32),
            pltpu.SemaphoreType.DMA,
            pltpu.SemaphoreType.DMA,
        ],
    )
    def gather(p0_hbm, src2_hbm, dst2_hbm,
               s_hbm, idxa, idxb, rows, tbl, semg, semw):
        sid = lax.axis_index("s")
        w = sid * _NC + lax.axis_index("c")
        nj = jnp.where(w < extra, base_nj + 1, base_nj)

        @pl.when(sid < _NS - 1)
        def _():
            pltpu.sync_copy(p0_hbm.at[pl.ds(sid * rps, rps)],
                            tbl.at[pl.ds(sid * rps, rps)])

        @pl.when(sid == _NS - 1)
        def _():
            pltpu.sync_copy(p0_hbm.at[pl.ds(sid * rps, last_rows)],
                            tbl.at[pl.ds(sid * rps, last_rows)])

        plsc.subcore_barrier()

        def body(j, carry):
            r = (w + _NW * j) * SC
            pltpu.sync_copy(src2_hbm.at[pl.ds(row0 + r, SC)], idxa)
            pltpu.sync_copy(dst2_hbm.at[pl.ds(row0 + r, SC)], idxb)
            wr = [None] * SC
            for t in range(SC):
                s = t % 2
                if t >= 2:
                    wr[t - 2].wait()
                pltpu.async_copy(tbl.at[idxa.at[t]], rows.at[s], semg).wait()
                pltpu.async_copy(tbl.at[idxb.at[t]], rows.at[s], semg,
                                 add=True).wait()
                wr[t] = pltpu.async_copy(
                    rows.at[s], s_hbm.at[pl.ds((r + t) * 128, 128)], semw)
            wr[SC - 2].wait()
            wr[SC - 1].wait()
            return carry

        lax.fori_loop(0, nj, body, 0)

        if tail_rows:
            @pl.when(w == 0)
            def _():
                r = n_super * SC
                pltpu.sync_copy(src2_hbm.at[pl.ds(row0 + r, tail_rows)],
                                idxa.at[pl.ds(0, tail_rows)])
                pltpu.sync_copy(dst2_hbm.at[pl.ds(row0 + r, tail_rows)],
                                idxb.at[pl.ds(0, tail_rows)])
                for t in range(tail_rows):
                    pltpu.async_copy(tbl.at[idxa.at[t]], rows.at[0],
                                     semg).wait()
                    pltpu.async_copy(tbl.at[idxb.at[t]], rows.at[0], semg,
                                     add=True).wait()
                    pltpu.sync_copy(rows.at[0],
                                    s_hbm.at[pl.ds((r + t) * 128, 128)])

    return gather


def _make_sc_scatter(N, H, row0, n_rows):
    SC = 8
    n_super = n_rows // SC
    tail_rows = n_rows - n_super * SC
    base_nj = n_super // _NW
    extra = n_super - base_nj * _NW
    rps = (N // _NS) // 8 * 8
    last_rows = N - rps * (_NS - 1)

    mesh = plsc.VectorSubcoreMesh(core_axis_name="c", subcore_axis_name="s",
                                  num_cores=_NC, num_subcores=_NS)

    @functools.partial(
        pl.kernel,
        out_type=jax.ShapeDtypeStruct((_NC, N, H), jnp.float32),
        mesh=mesh,
        scratch_types=[
            pltpu.VMEM((SC, 128), jnp.int32),
            pltpu.VMEM((2, 128, H), jnp.float32),
            pltpu.VMEM_SHARED((N, H), jnp.float32),
            pltpu.SemaphoreType.DMA,
            pltpu.SemaphoreType.DMA,
        ],
    )
    def scatter(tw_hbm, dst2_hbm, zeros_hbm, out_hbm, idxk, twbuf, acc,
                seml, sems):
        c = lax.axis_index("c")
        sid = lax.axis_index("s")
        w = sid * _NC + c

        @pl.when(sid < _NS - 1)
        def _():
            pltpu.sync_copy(zeros_hbm.at[pl.ds(0, rps)],
                            acc.at[pl.ds(sid * rps, rps)])

        @pl.when(sid == _NS - 1)
        def _():
            pltpu.sync_copy(zeros_hbm.at[pl.ds(0, last_rows)],
                            acc.at[pl.ds(sid * rps, last_rows)])

        plsc.subcore_barrier()

        nj = jnp.where(w < extra, base_nj + 1, base_nj)

        def body(j, carry):
            r = (w + _NW * j) * SC
            pltpu.sync_copy(dst2_hbm.at[pl.ds(row0 + r, SC)], idxk)
            ld = [None] * SC
            sc_ = [None] * SC

            def fire(t):
                ld[t] = pltpu.async_copy(
                    tw_hbm.at[pl.ds((r + t) * 128, 128)],
                    twbuf.at[t % 2], seml)

            fire(0)
            for t in range(SC):
                s = t % 2
                if t >= 1:
                    sc_[t - 1].wait()
                if t + 1 < SC:
                    fire(t + 1)
                ld[t].wait()
                sc_[t] = pltpu.async_copy(twbuf.at[s],
                                          acc.at[idxk.at[t]], sems,
                                          add=True)
            sc_[SC - 1].wait()
            return carry

        lax.fori_loop(0, nj, body, 0)

        if tail_rows:
            @pl.when(w == 0)
            def _():
                r = n_super * SC
                pltpu.sync_copy(dst2_hbm.at[pl.ds(row0 + r, tail_rows)],
                                idxk.at[pl.ds(0, tail_rows)])
                for t in range(tail_rows):
                    pltpu.sync_copy(tw_hbm.at[pl.ds((r + t) * 128, 128)],
                                    twbuf.at[0])
                    pltpu.sync_copy(twbuf.at[0],
                                    acc.at[idxk.at[t]], add=True)

        plsc.subcore_barrier()

        @pl.when(sid < _NS - 1)
        def _():
            pltpu.sync_copy(acc.at[pl.ds(sid * rps, rps)],
                            out_hbm.at[c, pl.ds(sid * rps, rps)])

        @pl.when(sid == _NS - 1)
        def _():
            pltpu.sync_copy(acc.at[pl.ds(sid * rps, last_rows)],
                            out_hbm.at[c, pl.ds(sid * rps, last_rows)])

    return scatter


def _edge_call(ea, s, We0, Wfb, bfe0, We1, e_base, E_s, be, DE, Hh, H):
    nb = e_base // be if e_base % be == 0 else None
    steps = E_s // be
    if nb is None:
        raise ValueError("slice base must be a multiple of the block size")
    return pl.pallas_call(
        _edge_body,
        grid=(steps,),
        in_specs=[
            pl.BlockSpec((be, DE), lambda i: (i + nb, 0)),
            pl.BlockSpec((be, H), lambda i: (i, 0)),
            _full((DE, Hh)),
            _full((Hh, H)),
            _full((1, H)),
            _full((H, Hh)),
        ],
        out_specs=pl.BlockSpec((be, H), lambda i: (i, 0)),
        out_shape=jax.ShapeDtypeStruct((E_s, H), jnp.float32),
    )(ea, s, We0, Wfb, bfe0, We1)


def kernel(x, edge_index, edge_attr, Wn0, We0, Wfn0, bfn0, Wfe0, bfe0,
           Wn1, We1, Wfn1, bfn1, Wfe1, bfe1):
    N, DN = x.shape
    E, DE = edge_attr.shape
    H = Wfn0.shape[0]
    Hh = H // 2
    src2 = edge_index[0].reshape(E // 128, 128)
    dst2 = edge_index[1].reshape(E // 128, 128)

    bn = 2000 if N % 2000 == 0 else N
    R = E // 128

    if E == 320000:
        rows_a = 1280
        be_a, be_b = 4096, 2560
    else:
        rows_a = (R // 16) * 8
        be_a = be_b = 128
    rows_b = R - rows_a
    E_a, E_b = rows_a * 128, rows_b * 128

    h0, p0 = pl.pallas_call(
        _h0p0_body,
        grid=(N // bn,),
        in_specs=[pl.BlockSpec((bn, DN), lambda i: (i, 0)), _full((DN, Hh)),
                  _full((Hh, H))],
        out_specs=[pl.BlockSpec((bn, Hh), lambda i: (i, 0)),
                   pl.BlockSpec((bn, H), lambda i: (i, 0))],
        out_shape=[jax.ShapeDtypeStruct((N, Hh), jnp.float32),
                   jax.ShapeDtypeStruct((N, H), jnp.float32)],
    )(x, Wn0, Wfe0[:Hh])

    s_a = _make_sc_gather(N, H, 0, rows_a)(p0, src2, dst2)
    s_b = _make_sc_gather(N, H, rows_a, rows_b)(p0, src2, dst2)

    zeros = jnp.zeros((N - (N // _NS) // 8 * 8 * (_NS - 1), H), jnp.float32)

    tw_a = _edge_call(edge_attr, s_a, We0, Wfe0[Hh:], bfe0[None], We1,
                      0, E_a, be_a, DE, Hh, H)
    part_a = _make_sc_scatter(N, H, 0, rows_a)(tw_a, dst2, zeros)

    tw_b = _edge_call(edge_attr, s_b, We0, Wfe0[Hh:], bfe0[None], We1,
                      E_a, E_b, be_b, DE, Hh, H)
    part_b = _make_sc_scatter(N, H, rows_a, rows_b)(tw_b, dst2, zeros)

    out = pl.pallas_call(
        _node_body,
        grid=(N // bn,),
        in_specs=[
            pl.BlockSpec((_NC, bn, H), lambda i: (0, i, 0)),
            pl.BlockSpec((_NC, bn, H), lambda i: (0, i, 0)),
            pl.BlockSpec((bn, Hh), lambda i: (i, 0)),
            _full((Hh, H)),
            _full((Hh, H)),
            _full((1, H)),
            _full((H, Hh)),
            _full((Hh, H)),
            _full((Hh, H)),
            _full((1, H)),
        ],
        out_specs=pl.BlockSpec((bn, H), lambda i: (i, 0)),
        out_shape=jax.ShapeDtypeStruct((N, H), jnp.float32),
    )(part_a, part_b, h0, Wfn0[:Hh], Wfn0[Hh:], bfn0[None], Wn1,
      Wfn1[:Hh], Wfn1[Hh:], bfn1[None])

    return out
